# baseline (speedup 1.0000x reference)
import functools

import jax
import jax.numpy as jnp
import numpy as np
from jax.experimental import pallas as pl

_LANES = 384
_PTS_PER_ROW = 128

_OFFS1 = np.tile(np.array([0.5, 0.0, 0.0], np.float32), _PTS_PER_ROW)[None, :]
_OFFS2 = np.tile(np.array([-0.5, -0.2, 0.0], np.float32), _PTS_PER_ROW)[None, :]
_LANE0 = np.tile(np.array([1.0, 0.0, 0.0], np.float32), _PTS_PER_ROW)[None, :]
_SEL = np.zeros((_LANES, _PTS_PER_ROW), np.float32)
_SEL[3 * np.arange(_PTS_PER_ROW), np.arange(_PTS_PER_ROW)] = 1
_CONSTS = np.concatenate([_OFFS1, _OFFS2, _LANE0], axis=0)


def _roll_m1(a):
    return jnp.concatenate([a[:, 1:], a[:, :1]], axis=1)


def _roll_m2(a):
    return jnp.concatenate([a[:, 2:], a[:, :2]], axis=1)


def _balls_kernel(pos_ref, const_ref, sel_ref, den_ref, rad_ref):
    v = pos_ref[...]
    offs1 = const_ref[0:1, :]
    offs2 = const_ref[1:2, :]
    lane0 = const_ref[2:3, :]

    d1 = v - offs1
    s1 = d1 * d1
    q1 = (s1 + _roll_m1(s1)) + _roll_m2(s1)
    in1 = q1 < 0.3

    d2 = v - offs2
    s2 = d2 * d2
    q2 = (s2 + _roll_m1(s2)) + _roll_m2(s2)
    in2 = q2 < 0.8

    inside = in1 | in2
    rad = jnp.where(inside, 1.0, 0.0).astype(jnp.float32) * lane0
    rad_ref[...] = rad
    den_ref[...] = jnp.dot(
        rad.astype(jnp.bfloat16), sel_ref[...], preferred_element_type=jnp.float32
    )


@functools.partial(jax.jit, static_argnames=())
def _run(position):
    n = position.shape[0]
    rows = n // _PTS_PER_ROW
    block_rows = 512
    grid = rows // block_rows
    pos2d = position.reshape(rows, _LANES)
    consts = jnp.asarray(_CONSTS)
    sel = jnp.asarray(_SEL, dtype=jnp.bfloat16)
    den, rad = pl.pallas_call(
        _balls_kernel,
        grid=(grid,),
        in_specs=[
            pl.BlockSpec((block_rows, _LANES), lambda i: (i, 0)),
            pl.BlockSpec((3, _LANES), lambda i: (0, 0)),
            pl.BlockSpec((_LANES, _PTS_PER_ROW), lambda i: (0, 0)),
        ],
        out_specs=[
            pl.BlockSpec((block_rows, _PTS_PER_ROW), lambda i: (i, 0)),
            pl.BlockSpec((block_rows, _LANES), lambda i: (i, 0)),
        ],
        out_shape=[
            jax.ShapeDtypeStruct((rows, _PTS_PER_ROW), jnp.float32),
            jax.ShapeDtypeStruct((rows, _LANES), jnp.float32),
        ],
    )(pos2d, consts, sel)
    return den.reshape(n), rad.reshape(n, 3)


def kernel(position, direction):
    del direction
    return _run(position)


# speedup vs baseline: 105.9225x; 105.9225x over previous
import functools

import jax
import jax.numpy as jnp
from jax.experimental import pallas as pl


def _balls_kernel(pos_ref, den_ref, rad_ref):
    x = pos_ref[0:1, :]
    y = pos_ref[1:2, :]
    z = pos_ref[2:3, :]

    zz = z * z
    q1 = (jnp.square(x - 0.5) + jnp.square(y)) + zz
    q2 = (jnp.square(x + 0.5) + jnp.square(y + 0.2)) + zz
    inside = (q1 < 0.3) | (q2 < 0.8)

    m = jnp.where(inside, jnp.float32(1.0), jnp.float32(0.0))
    den_ref[...] = m
    rad_ref[0:1, :] = m
    rad_ref[1:3, :] = jnp.zeros_like(pos_ref[1:3, :])


@jax.jit
def _run(position):
    n = position.shape[0]
    lanes = 65536
    grid = n // lanes
    pos_t = position.T
    den, rad = pl.pallas_call(
        _balls_kernel,
        grid=(grid,),
        in_specs=[pl.BlockSpec((3, lanes), lambda i: (0, i))],
        out_specs=[
            pl.BlockSpec((1, lanes), lambda i: (0, i)),
            pl.BlockSpec((3, lanes), lambda i: (0, i)),
        ],
        out_shape=[
            jax.ShapeDtypeStruct((1, n), jnp.float32),
            jax.ShapeDtypeStruct((3, n), jnp.float32),
        ],
    )(pos_t)
    return den.reshape(n), rad.T


def kernel(position, direction):
    del direction
    return _run(position)
